# baseline (speedup 1.0000x reference)
import functools

import numpy as np

import jax
import jax.numpy as jnp
from jax import lax
from jax.experimental import pallas as pl
from jax.experimental.pallas import tpu as pltpu
from jax.experimental.pallas import tpu_sc as plsc

N_PROPOSALS = 20000
M_GT = 100
N_TOT = N_PROPOSALS + M_GT
NUM_CLASSES = 80
NUM_FG = 128
NUM_BG = 384
NUM_SAMPLES = NUM_FG + NUM_BG

LANES = 16
N_PAD = 20480
N_WORKERS = 32
CHUNK = N_PAD // N_WORKERS
VPC = CHUNK // LANES
M_PAD = 128

TC_ROWS = 14336
SC_ROWS = N_PAD - TC_ROWS
SC_CHUNK = SC_ROWS // N_WORKERS
SC_VPC = SC_CHUNK // LANES

_MESH = plsc.VectorSubcoreMesh(core_axis_name="c", subcore_axis_name="s")


def _sampling_perm() -> np.ndarray:
    mask = np.uint32(0xFFFFFFFF)

    def rotl(x, d):
        return ((x << np.uint32(d)) | (x >> np.uint32(32 - d))) & mask

    ks = [np.uint32(0), np.uint32(42),
          np.uint32(0) ^ np.uint32(42) ^ np.uint32(0x1BD11BDA)]
    rot = [13, 15, 26, 6, 17, 29, 16, 24]
    x0 = np.zeros(N_TOT, np.uint32) + ks[0]
    x1 = (np.arange(N_TOT, dtype=np.uint32) + ks[1]) & mask
    for i in range(5):
        for r in (rot[0:4] if i % 2 == 0 else rot[4:8]):
            x0 = (x0 + x1) & mask
            x1 = rotl(x1, r)
            x1 = x1 ^ x0
        x0 = (x0 + ks[(i + 1) % 3]) & mask
        x1 = (x1 + ks[(i + 2) % 3] + np.uint32(i + 1)) & mask
    bits = x0 ^ x1
    scores = (((bits >> np.uint32(9)) | np.uint32(0x3F800000))
              .view(np.float32) - np.float32(1.0))
    perm = np.argsort(-scores, kind="stable").astype(np.int32)
    return np.concatenate([perm, np.arange(N_TOT, N_PAD, dtype=np.int32)])


_PERM = _sampling_perm()


def _wid():
    return lax.axis_index("s") * 2 + lax.axis_index("c")


def _iou_cls_body(px0, py0, px1, py1, gx0, gy0, gx1, gy1, gcls,
                  iou_out, cls_out,
                  vx0, vy0, vx1, vy1, vg0, vg1, vg2, vg3, vgc,
                  viou, vcls):
    wid = _wid()
    base = TC_ROWS + wid * SC_CHUNK
    pltpu.sync_copy(px0.at[pl.ds(base, SC_CHUNK)], vx0)
    pltpu.sync_copy(py0.at[pl.ds(base, SC_CHUNK)], vy0)
    pltpu.sync_copy(px1.at[pl.ds(base, SC_CHUNK)], vx1)
    pltpu.sync_copy(py1.at[pl.ds(base, SC_CHUNK)], vy1)
    pltpu.sync_copy(gx0, vg0)
    pltpu.sync_copy(gy0, vg1)
    pltpu.sync_copy(gx1, vg2)
    pltpu.sync_copy(gy1, vg3)
    pltpu.sync_copy(gcls, vgc)

    def per_pair(i, _):
        sl0 = pl.ds((2 * i) * LANES, LANES)
        sl1 = pl.ds((2 * i + 1) * LANES, LANES)
        x0a = vx0[sl0]
        y0a = vy0[sl0]
        x1a = vx1[sl0]
        y1a = vy1[sl0]
        x0b = vx0[sl1]
        y0b = vy0[sl1]
        x1b = vx1[sl1]
        y1b = vy1[sl1]
        parea_a = (x1a - x0a) * (y1a - y0a)
        parea_b = (x1b - x0b) * (y1b - y0b)

        def one(a0, a1, a2, a3, garea, gc, x0, y0, x1, y1, parea,
                best, bcls):
            ltx = jnp.maximum(a0, x0)
            lty = jnp.maximum(a1, y0)
            rbx = jnp.minimum(a2, x1)
            rby = jnp.minimum(a3, y1)
            w = jnp.maximum(rbx - ltx, 0.0)
            h = jnp.maximum(rby - lty, 0.0)
            inter = w * h
            union = garea + parea - inter
            iou = jnp.where(inter > 0.0,
                            inter / jnp.maximum(union, 1e-8),
                            0.0)
            upd = iou > best
            return jnp.where(upd, iou, best), jnp.where(upd, gc, bcls)

        def per_gt(g, carry):
            best_a, bcls_a, best_b, bcls_b = carry
            gi = jnp.full((LANES,), g, dtype=jnp.int32)
            a0 = plsc.load_gather(vg0, [gi])
            a1 = plsc.load_gather(vg1, [gi])
            a2 = plsc.load_gather(vg2, [gi])
            a3 = plsc.load_gather(vg3, [gi])
            gc = plsc.load_gather(vgc, [gi])
            garea = (a2 - a0) * (a3 - a1)
            best_a, bcls_a = one(a0, a1, a2, a3, garea, gc,
                                 x0a, y0a, x1a, y1a, parea_a,
                                 best_a, bcls_a)
            best_b, bcls_b = one(a0, a1, a2, a3, garea, gc,
                                 x0b, y0b, x1b, y1b, parea_b,
                                 best_b, bcls_b)
            return best_a, bcls_a, best_b, bcls_b

        neg = jnp.full((LANES,), -1.0, dtype=jnp.float32)
        zero = jnp.zeros((LANES,), dtype=jnp.int32)
        best_a, bcls_a, best_b, bcls_b = lax.fori_loop(
            0, M_GT, per_gt, (neg, zero, neg, zero))
        bg = jnp.full((LANES,), NUM_CLASSES, dtype=jnp.int32)
        viou[sl0] = best_a
        vcls[sl0] = jnp.where(best_a >= 0.5, bcls_a, bg)
        viou[sl1] = best_b
        vcls[sl1] = jnp.where(best_b >= 0.5, bcls_b, bg)
        return 0

    lax.fori_loop(0, SC_VPC // 2, per_pair, 0)
    pltpu.sync_copy(viou, iou_out.at[pl.ds(wid * SC_CHUNK, SC_CHUNK)])
    pltpu.sync_copy(vcls, cls_out.at[pl.ds(wid * SC_CHUNK, SC_CHUNK)])


_iou_cls_kernel = functools.partial(
    pl.kernel, _iou_cls_body, mesh=_MESH,
    compiler_params=pltpu.CompilerParams(needs_layout_passes=False),
    out_type=[jax.ShapeDtypeStruct((SC_ROWS,), jnp.float32),
              jax.ShapeDtypeStruct((SC_ROWS,), jnp.int32)],
    scratch_types=[pltpu.VMEM((SC_CHUNK,), jnp.float32)] * 4
                  + [pltpu.VMEM((M_PAD,), jnp.float32)] * 4
                  + [pltpu.VMEM((M_PAD,), jnp.int32)]
                  + [pltpu.VMEM((SC_CHUNK,), jnp.float32),
                     pltpu.VMEM((SC_CHUNK,), jnp.int32)],
)


def _tc_iou_body(gx0s, gy0s, gx1s, gy1s, gcs,
                 px0b, py0b, px1b, py1b, iou_b, cls_b):
    x0 = px0b[...]
    y0 = py0b[...]
    x1 = px1b[...]
    y1 = py1b[...]
    parea = (x1 - x0) * (y1 - y0)

    def per_gt(g, carry):
        best, bcls = carry
        a0 = gx0s[g]
        a1 = gy0s[g]
        a2 = gx1s[g]
        a3 = gy1s[g]
        gc = gcs[g]
        garea = (a2 - a0) * (a3 - a1)
        ltx = jnp.maximum(x0, a0)
        lty = jnp.maximum(y0, a1)
        rbx = jnp.minimum(x1, a2)
        rby = jnp.minimum(y1, a3)
        w = jnp.maximum(rbx - ltx, 0.0)
        h = jnp.maximum(rby - lty, 0.0)
        inter = w * h
        union = garea + parea - inter
        iou = jnp.where(inter > 0.0,
                        inter / jnp.maximum(union, 1e-8),
                        0.0)
        upd = iou > best
        return jnp.where(upd, iou, best), jnp.where(upd, gc, bcls)

    init = (jnp.full(x0.shape, -1.0, dtype=jnp.float32),
            jnp.zeros(x0.shape, dtype=jnp.int32))
    best, bcls = lax.fori_loop(0, M_GT, per_gt, init)
    iou_b[...] = best
    cls_b[...] = jnp.where(best >= 0.5, bcls, NUM_CLASSES)


def _tc_iou(px0, py0, px1, py1, gx0, gy0, gx1, gy1, gcls):
    rows = TC_ROWS // 128
    grid = rows // 8
    smem = pl.BlockSpec(memory_space=pltpu.SMEM)
    blk = pl.BlockSpec((8, 128), lambda i: (i, 0))
    iou, cls = pl.pallas_call(
        _tc_iou_body,
        grid=(grid,),
        in_specs=[smem] * 5 + [blk] * 4,
        out_specs=[blk, blk],
        out_shape=[jax.ShapeDtypeStruct((rows, 128), jnp.float32),
                   jax.ShapeDtypeStruct((rows, 128), jnp.int32)],
    )(gx0, gy0, gx1, gy1, gcls,
      px0[:TC_ROWS].reshape(rows, 128), py0[:TC_ROWS].reshape(rows, 128),
      px1[:TC_ROWS].reshape(rows, 128), py1[:TC_ROWS].reshape(rows, 128))
    return iou.reshape(TC_ROWS), cls.reshape(TC_ROWS)


def _select_body(cls_hbm, perm_hbm, px0, py0, px1, py1,
                 idx_out, scls_out, sbox_out,
                 vcls, vx0, vy0, vx1, vy1, vperm,
                 vidx, vocls, vob0, vob1, vob2, vob3, sem_c, sem_b):
    wid = _wid()

    @pl.when(wid == 0)
    def _():
        h_cls = pltpu.async_copy(cls_hbm, vcls, sem_c)
        h0 = pltpu.async_copy(px0, vx0, sem_b)
        h1 = pltpu.async_copy(py0, vy0, sem_b)
        h2 = pltpu.async_copy(px1, vx1, sem_b)
        h3 = pltpu.async_copy(py1, vy1, sem_b)
        h_cls.wait()

        zero = jnp.zeros((LANES,), jnp.int32)

        def chunk_cond(carry):
            c, nfgv, nbgv = carry
            return jnp.logical_and(
                c < N_WORKERS,
                jnp.logical_or(jnp.max(nfgv) < NUM_FG,
                               jnp.max(nbgv) < NUM_BG))

        def per_chunk(carry):
            c, nfgv, nbgv = carry
            pltpu.sync_copy(perm_hbm.at[pl.ds(c * CHUNK, CHUNK)], vperm)

            def per_vreg(j, carry2):
                nfgv, nbgv = carry2
                pv = vperm[pl.ds(j * LANES, LANES)]
                cv = plsc.load_gather(vcls, [pv])
                valid = pv < N_TOT
                fgm = jnp.logical_and(cv < NUM_CLASSES, valid)
                bgm = jnp.logical_and(cv == NUM_CLASSES, valid)
                cumf = nfgv + jnp.cumsum(fgm.astype(jnp.int32))
                cumb = nbgv + jnp.cumsum(bgm.astype(jnp.int32))
                self_f = jnp.logical_and(fgm, cumf <= NUM_FG)
                self_b = jnp.logical_and(bgm, cumb <= NUM_BG)
                plsc.store_scatter(vidx, [cumf - 1], pv, mask=self_f)
                plsc.store_scatter(vidx, [NUM_FG + cumb - 1], pv,
                                   mask=self_b)
                nfgv = nfgv + plsc.all_reduce_population_count(fgm)
                nbgv = nbgv + plsc.all_reduce_population_count(bgm)
                return nfgv, nbgv

            nfgv, nbgv = lax.fori_loop(0, VPC, per_vreg, (nfgv, nbgv))
            return c + 1, nfgv, nbgv

        _, nfgv, nbgv = lax.while_loop(
            chunk_cond, per_chunk, (jnp.int32(0), zero, zero))
        nfg = jnp.max(nfgv)
        nbg = jnp.max(nbgv)

        @pl.when(nfg < NUM_FG)
        def _():
            def fill_f(j, n):
                sl = pl.ds(j * LANES, LANES)
                cv = vcls[sl]
                iv = j * LANES + lax.iota(jnp.int32, LANES)
                m = jnp.logical_and(cv == NUM_CLASSES, iv < N_TOT)
                cum = n + jnp.cumsum(m.astype(jnp.int32))
                sel = jnp.logical_and(m, cum <= NUM_FG)
                plsc.store_scatter(vidx, [cum - 1], iv, mask=sel)
                return jnp.max(cum)

            lax.fori_loop(0, N_PAD // LANES, fill_f, nfg)

        @pl.when(nbg < NUM_BG)
        def _():
            def fill_b(j, n):
                sl = pl.ds(j * LANES, LANES)
                cv = vcls[sl]
                iv = j * LANES + lax.iota(jnp.int32, LANES)
                m = jnp.logical_and(cv < NUM_CLASSES, iv < N_TOT)
                cum = n + jnp.cumsum(m.astype(jnp.int32))
                sel = jnp.logical_and(m, cum <= NUM_BG)
                plsc.store_scatter(vidx, [NUM_FG + cum - 1], iv, mask=sel)
                return jnp.max(cum)

            lax.fori_loop(0, N_PAD // LANES, fill_b, nbg)

        h0.wait()
        h1.wait()
        h2.wait()
        h3.wait()

        def gather_out(k, _):
            sl = pl.ds(k * LANES, LANES)
            iv = vidx[sl]
            vocls[sl] = plsc.load_gather(vcls, [iv])
            vob0[sl] = plsc.load_gather(vx0, [iv])
            vob1[sl] = plsc.load_gather(vy0, [iv])
            vob2[sl] = plsc.load_gather(vx1, [iv])
            vob3[sl] = plsc.load_gather(vy1, [iv])
            return 0

        lax.fori_loop(0, NUM_SAMPLES // LANES, gather_out, 0)
        pltpu.sync_copy(vidx, idx_out)
        pltpu.sync_copy(vocls, scls_out)
        pltpu.sync_copy(vob0, sbox_out.at[0])
        pltpu.sync_copy(vob1, sbox_out.at[1])
        pltpu.sync_copy(vob2, sbox_out.at[2])
        pltpu.sync_copy(vob3, sbox_out.at[3])


_select_kernel = functools.partial(
    pl.kernel, _select_body, mesh=_MESH,
    compiler_params=pltpu.CompilerParams(needs_layout_passes=False),
    out_type=[jax.ShapeDtypeStruct((NUM_SAMPLES,), jnp.int32),
              jax.ShapeDtypeStruct((NUM_SAMPLES,), jnp.int32),
              jax.ShapeDtypeStruct((4, NUM_SAMPLES), jnp.float32)],
    scratch_types=[pltpu.VMEM((N_PAD,), jnp.int32)]
                  + [pltpu.VMEM((N_PAD,), jnp.float32)] * 4
                  + [pltpu.VMEM((CHUNK,), jnp.int32)]
                  + [pltpu.VMEM((NUM_SAMPLES,), jnp.int32)] * 2
                  + [pltpu.VMEM((NUM_SAMPLES,), jnp.float32)] * 4
                  + [pltpu.SemaphoreType.DMA] * 2,
)


def kernel(proposal_boxes, gt_boxes, gt_classes):
    props = jnp.concatenate(
        [proposal_boxes.astype(jnp.float32), gt_boxes.astype(jnp.float32)],
        axis=0)
    props_pad = jnp.pad(props, ((0, N_PAD - N_TOT), (0, 0)))
    px0, py0, px1, py1 = [props_pad[:, i] for i in range(4)]
    gpad = jnp.pad(gt_boxes.astype(jnp.float32), ((0, M_PAD - M_GT), (0, 0)))
    gx0, gy0, gx1, gy1 = [gpad[:, i] for i in range(4)]
    gcls = jnp.pad(gt_classes.astype(jnp.int32), (0, M_PAD - M_GT),
                   constant_values=NUM_CLASSES)

    perm = jnp.asarray(_PERM)

    iou_sc, cls_sc = _iou_cls_kernel()(
        px0, py0, px1, py1, gx0, gy0, gx1, gy1, gcls)
    iou_tc, cls_tc = _tc_iou(px0, py0, px1, py1, gx0, gy0, gx1, gy1, gcls)
    iou_pad = jnp.concatenate([iou_tc, iou_sc])
    cls_pad = jnp.concatenate([cls_tc, cls_sc])
    sampled_idxs, sampled_classes, sbox = _select_kernel()(
        cls_pad, perm, px0, py0, px1, py1)

    sampled_boxes = sbox.T
    iou_with_gt = iou_pad[:N_TOT]
    return sampled_idxs, sampled_classes, sampled_boxes, iou_with_gt


# speedup vs baseline: 1.2975x; 1.2975x over previous
import functools

import numpy as np

import jax
import jax.numpy as jnp
from jax import lax
from jax.experimental import pallas as pl
from jax.experimental.pallas import tpu as pltpu
from jax.experimental.pallas import tpu_sc as plsc

N_PROPOSALS = 20000
M_GT = 100
N_TOT = N_PROPOSALS + M_GT
NUM_CLASSES = 80
NUM_FG = 128
NUM_BG = 384
NUM_SAMPLES = NUM_FG + NUM_BG

LANES = 16
N_PAD = 20480
N_WORKERS = 32
CHUNK = N_PAD // N_WORKERS
VPC = CHUNK // LANES
M_PAD = 128

TC_ROWS = 12288
SC_ROWS = N_PAD - TC_ROWS
SC_CHUNK = SC_ROWS // N_WORKERS
SC_VPC = SC_CHUNK // LANES

_MESH = plsc.VectorSubcoreMesh(core_axis_name="c", subcore_axis_name="s")


def _sampling_perm() -> np.ndarray:
    mask = np.uint32(0xFFFFFFFF)

    def rotl(x, d):
        return ((x << np.uint32(d)) | (x >> np.uint32(32 - d))) & mask

    ks = [np.uint32(0), np.uint32(42),
          np.uint32(0) ^ np.uint32(42) ^ np.uint32(0x1BD11BDA)]
    rot = [13, 15, 26, 6, 17, 29, 16, 24]
    x0 = np.zeros(N_TOT, np.uint32) + ks[0]
    x1 = (np.arange(N_TOT, dtype=np.uint32) + ks[1]) & mask
    for i in range(5):
        for r in (rot[0:4] if i % 2 == 0 else rot[4:8]):
            x0 = (x0 + x1) & mask
            x1 = rotl(x1, r)
            x1 = x1 ^ x0
        x0 = (x0 + ks[(i + 1) % 3]) & mask
        x1 = (x1 + ks[(i + 2) % 3] + np.uint32(i + 1)) & mask
    bits = x0 ^ x1
    scores = (((bits >> np.uint32(9)) | np.uint32(0x3F800000))
              .view(np.float32) - np.float32(1.0))
    perm = np.argsort(-scores, kind="stable").astype(np.int32)
    return np.concatenate([perm, np.arange(N_TOT, N_PAD, dtype=np.int32)])


_PERM = _sampling_perm()


def _wid():
    return lax.axis_index("s") * 2 + lax.axis_index("c")


def _iou_cls_body(px0, py0, px1, py1, gx0, gy0, gx1, gy1, gcls,
                  iou_out, cls_out,
                  vx0, vy0, vx1, vy1, vg0, vg1, vg2, vg3, vgc,
                  viou, vcls):
    wid = _wid()
    base = TC_ROWS + wid * SC_CHUNK
    pltpu.sync_copy(px0.at[pl.ds(base, SC_CHUNK)], vx0)
    pltpu.sync_copy(py0.at[pl.ds(base, SC_CHUNK)], vy0)
    pltpu.sync_copy(px1.at[pl.ds(base, SC_CHUNK)], vx1)
    pltpu.sync_copy(py1.at[pl.ds(base, SC_CHUNK)], vy1)
    pltpu.sync_copy(gx0, vg0)
    pltpu.sync_copy(gy0, vg1)
    pltpu.sync_copy(gx1, vg2)
    pltpu.sync_copy(gy1, vg3)
    pltpu.sync_copy(gcls, vgc)

    def per_pair(i, _):
        sl0 = pl.ds((2 * i) * LANES, LANES)
        sl1 = pl.ds((2 * i + 1) * LANES, LANES)
        x0a = vx0[sl0]
        y0a = vy0[sl0]
        x1a = vx1[sl0]
        y1a = vy1[sl0]
        x0b = vx0[sl1]
        y0b = vy0[sl1]
        x1b = vx1[sl1]
        y1b = vy1[sl1]
        parea_a = (x1a - x0a) * (y1a - y0a)
        parea_b = (x1b - x0b) * (y1b - y0b)

        def one(a0, a1, a2, a3, garea, gc, x0, y0, x1, y1, parea,
                best, bcls):
            ltx = jnp.maximum(a0, x0)
            lty = jnp.maximum(a1, y0)
            rbx = jnp.minimum(a2, x1)
            rby = jnp.minimum(a3, y1)
            w = jnp.maximum(rbx - ltx, 0.0)
            h = jnp.maximum(rby - lty, 0.0)
            inter = w * h
            union = garea + parea - inter
            iou = jnp.where(inter > 0.0,
                            inter / jnp.maximum(union, 1e-8),
                            0.0)
            upd = iou > best
            return jnp.where(upd, iou, best), jnp.where(upd, gc, bcls)

        def per_gt(g, carry):
            best_a, bcls_a, best_b, bcls_b = carry
            gi = jnp.full((LANES,), g, dtype=jnp.int32)
            a0 = plsc.load_gather(vg0, [gi])
            a1 = plsc.load_gather(vg1, [gi])
            a2 = plsc.load_gather(vg2, [gi])
            a3 = plsc.load_gather(vg3, [gi])
            gc = plsc.load_gather(vgc, [gi])
            garea = (a2 - a0) * (a3 - a1)
            best_a, bcls_a = one(a0, a1, a2, a3, garea, gc,
                                 x0a, y0a, x1a, y1a, parea_a,
                                 best_a, bcls_a)
            best_b, bcls_b = one(a0, a1, a2, a3, garea, gc,
                                 x0b, y0b, x1b, y1b, parea_b,
                                 best_b, bcls_b)
            return best_a, bcls_a, best_b, bcls_b

        neg = jnp.full((LANES,), -1.0, dtype=jnp.float32)
        zero = jnp.zeros((LANES,), dtype=jnp.int32)
        best_a, bcls_a, best_b, bcls_b = lax.fori_loop(
            0, M_GT, per_gt, (neg, zero, neg, zero))
        bg = jnp.full((LANES,), NUM_CLASSES, dtype=jnp.int32)
        viou[sl0] = best_a
        vcls[sl0] = jnp.where(best_a >= 0.5, bcls_a, bg)
        viou[sl1] = best_b
        vcls[sl1] = jnp.where(best_b >= 0.5, bcls_b, bg)
        return 0

    lax.fori_loop(0, SC_VPC // 2, per_pair, 0)
    pltpu.sync_copy(viou, iou_out.at[pl.ds(wid * SC_CHUNK, SC_CHUNK)])
    pltpu.sync_copy(vcls, cls_out.at[pl.ds(wid * SC_CHUNK, SC_CHUNK)])


_iou_cls_kernel = functools.partial(
    pl.kernel, _iou_cls_body, mesh=_MESH,
    compiler_params=pltpu.CompilerParams(needs_layout_passes=False),
    out_type=[jax.ShapeDtypeStruct((SC_ROWS,), jnp.float32),
              jax.ShapeDtypeStruct((SC_ROWS,), jnp.int32)],
    scratch_types=[pltpu.VMEM((SC_CHUNK,), jnp.float32)] * 4
                  + [pltpu.VMEM((M_PAD,), jnp.float32)] * 4
                  + [pltpu.VMEM((M_PAD,), jnp.int32)]
                  + [pltpu.VMEM((SC_CHUNK,), jnp.float32),
                     pltpu.VMEM((SC_CHUNK,), jnp.int32)],
)


def _tc_iou_body(gx0s, gy0s, gx1s, gy1s, gcs,
                 px0b, py0b, px1b, py1b, iou_b, cls_b):
    x0 = px0b[...]
    y0 = py0b[...]
    x1 = px1b[...]
    y1 = py1b[...]
    parea = (x1 - x0) * (y1 - y0)

    def one_gt(g, best, bcls):
        a0 = gx0s[g]
        a1 = gy0s[g]
        a2 = gx1s[g]
        a3 = gy1s[g]
        gc = gcs[g]
        garea = (a2 - a0) * (a3 - a1)
        ltx = jnp.maximum(x0, a0)
        lty = jnp.maximum(y0, a1)
        rbx = jnp.minimum(x1, a2)
        rby = jnp.minimum(y1, a3)
        w = jnp.maximum(rbx - ltx, 0.0)
        h = jnp.maximum(rby - lty, 0.0)
        inter = w * h
        union = garea + parea - inter
        iou = jnp.where(inter > 0.0,
                        inter / jnp.maximum(union, 1e-8),
                        0.0)
        upd = iou > best
        return jnp.where(upd, iou, best), jnp.where(upd, gc, bcls)

    def per_gt4(q, carry):
        best, bcls = carry
        g = q * 4
        for k in range(4):
            best, bcls = one_gt(g + k, best, bcls)
        return best, bcls

    init = (jnp.full(x0.shape, -1.0, dtype=jnp.float32),
            jnp.zeros(x0.shape, dtype=jnp.int32))
    best, bcls = lax.fori_loop(0, M_GT // 4, per_gt4, init)
    iou_b[...] = best
    cls_b[...] = jnp.where(best >= 0.5, bcls, NUM_CLASSES)


def _tc_iou(px0, py0, px1, py1, gx0, gy0, gx1, gy1, gcls):
    rows = TC_ROWS // 128
    grid = rows // 8
    smem = pl.BlockSpec(memory_space=pltpu.SMEM)
    blk = pl.BlockSpec((8, 128), lambda i: (i, 0))
    iou, cls = pl.pallas_call(
        _tc_iou_body,
        grid=(grid,),
        in_specs=[smem] * 5 + [blk] * 4,
        out_specs=[blk, blk],
        out_shape=[jax.ShapeDtypeStruct((rows, 128), jnp.float32),
                   jax.ShapeDtypeStruct((rows, 128), jnp.int32)],
    )(gx0, gy0, gx1, gy1, gcls,
      px0[:TC_ROWS].reshape(rows, 128), py0[:TC_ROWS].reshape(rows, 128),
      px1[:TC_ROWS].reshape(rows, 128), py1[:TC_ROWS].reshape(rows, 128))
    return iou.reshape(TC_ROWS), cls.reshape(TC_ROWS)


def _select_body(cls_hbm, perm_hbm, px0, py0, px1, py1,
                 idx_out, scls_out, sbox_out,
                 vcls, vx0, vy0, vx1, vy1, vperm,
                 vidx, vocls, vob0, vob1, vob2, vob3, sem_c, sem_b):
    wid = _wid()

    @pl.when(wid == 0)
    def _():
        h_cls = pltpu.async_copy(cls_hbm, vcls, sem_c)
        h0 = pltpu.async_copy(px0, vx0, sem_b)
        h1 = pltpu.async_copy(py0, vy0, sem_b)
        h2 = pltpu.async_copy(px1, vx1, sem_b)
        h3 = pltpu.async_copy(py1, vy1, sem_b)
        h_cls.wait()

        zero = jnp.zeros((LANES,), jnp.int32)

        def chunk_cond(carry):
            c, nfgv, nbgv = carry
            return jnp.logical_and(
                c < N_WORKERS,
                jnp.logical_or(jnp.max(nfgv) < NUM_FG,
                               jnp.max(nbgv) < NUM_BG))

        def per_chunk(carry):
            c, nfgv, nbgv = carry
            pltpu.sync_copy(perm_hbm.at[pl.ds(c * CHUNK, CHUNK)], vperm)

            def per_vreg(j, carry2):
                nfgv, nbgv = carry2
                pv = vperm[pl.ds(j * LANES, LANES)]
                cv = plsc.load_gather(vcls, [pv])
                valid = pv < N_TOT
                fgm = jnp.logical_and(cv < NUM_CLASSES, valid)
                bgm = jnp.logical_and(cv == NUM_CLASSES, valid)
                cumf = nfgv + jnp.cumsum(fgm.astype(jnp.int32))
                cumb = nbgv + jnp.cumsum(bgm.astype(jnp.int32))
                self_f = jnp.logical_and(fgm, cumf <= NUM_FG)
                self_b = jnp.logical_and(bgm, cumb <= NUM_BG)
                plsc.store_scatter(vidx, [cumf - 1], pv, mask=self_f)
                plsc.store_scatter(vidx, [NUM_FG + cumb - 1], pv,
                                   mask=self_b)
                nfgv = nfgv + plsc.all_reduce_population_count(fgm)
                nbgv = nbgv + plsc.all_reduce_population_count(bgm)
                return nfgv, nbgv

            nfgv, nbgv = lax.fori_loop(0, VPC, per_vreg, (nfgv, nbgv))
            return c + 1, nfgv, nbgv

        _, nfgv, nbgv = lax.while_loop(
            chunk_cond, per_chunk, (jnp.int32(0), zero, zero))
        nfg = jnp.max(nfgv)
        nbg = jnp.max(nbgv)

        @pl.when(nfg < NUM_FG)
        def _():
            def fill_f(j, n):
                sl = pl.ds(j * LANES, LANES)
                cv = vcls[sl]
                iv = j * LANES + lax.iota(jnp.int32, LANES)
                m = jnp.logical_and(cv == NUM_CLASSES, iv < N_TOT)
                cum = n + jnp.cumsum(m.astype(jnp.int32))
                sel = jnp.logical_and(m, cum <= NUM_FG)
                plsc.store_scatter(vidx, [cum - 1], iv, mask=sel)
                return jnp.max(cum)

            lax.fori_loop(0, N_PAD // LANES, fill_f, nfg)

        @pl.when(nbg < NUM_BG)
        def _():
            def fill_b(j, n):
                sl = pl.ds(j * LANES, LANES)
                cv = vcls[sl]
                iv = j * LANES + lax.iota(jnp.int32, LANES)
                m = jnp.logical_and(cv < NUM_CLASSES, iv < N_TOT)
                cum = n + jnp.cumsum(m.astype(jnp.int32))
                sel = jnp.logical_and(m, cum <= NUM_BG)
                plsc.store_scatter(vidx, [NUM_FG + cum - 1], iv, mask=sel)
                return jnp.max(cum)

            lax.fori_loop(0, N_PAD // LANES, fill_b, nbg)

        h0.wait()
        h1.wait()
        h2.wait()
        h3.wait()

        def gather_out(k, _):
            sl = pl.ds(k * LANES, LANES)
            iv = vidx[sl]
            vocls[sl] = plsc.load_gather(vcls, [iv])
            vob0[sl] = plsc.load_gather(vx0, [iv])
            vob1[sl] = plsc.load_gather(vy0, [iv])
            vob2[sl] = plsc.load_gather(vx1, [iv])
            vob3[sl] = plsc.load_gather(vy1, [iv])
            return 0

        lax.fori_loop(0, NUM_SAMPLES // LANES, gather_out, 0)
        pltpu.sync_copy(vidx, idx_out)
        pltpu.sync_copy(vocls, scls_out)
        pltpu.sync_copy(vob0, sbox_out.at[0])
        pltpu.sync_copy(vob1, sbox_out.at[1])
        pltpu.sync_copy(vob2, sbox_out.at[2])
        pltpu.sync_copy(vob3, sbox_out.at[3])


_select_kernel = functools.partial(
    pl.kernel, _select_body, mesh=_MESH,
    compiler_params=pltpu.CompilerParams(needs_layout_passes=False),
    out_type=[jax.ShapeDtypeStruct((NUM_SAMPLES,), jnp.int32),
              jax.ShapeDtypeStruct((NUM_SAMPLES,), jnp.int32),
              jax.ShapeDtypeStruct((4, NUM_SAMPLES), jnp.float32)],
    scratch_types=[pltpu.VMEM((N_PAD,), jnp.int32)]
                  + [pltpu.VMEM((N_PAD,), jnp.float32)] * 4
                  + [pltpu.VMEM((CHUNK,), jnp.int32)]
                  + [pltpu.VMEM((NUM_SAMPLES,), jnp.int32)] * 2
                  + [pltpu.VMEM((NUM_SAMPLES,), jnp.float32)] * 4
                  + [pltpu.SemaphoreType.DMA] * 2,
)


def kernel(proposal_boxes, gt_boxes, gt_classes):
    props = jnp.concatenate(
        [proposal_boxes.astype(jnp.float32), gt_boxes.astype(jnp.float32)],
        axis=0)
    props_pad = jnp.pad(props, ((0, N_PAD - N_TOT), (0, 0)))
    px0, py0, px1, py1 = [props_pad[:, i] for i in range(4)]
    gpad = jnp.pad(gt_boxes.astype(jnp.float32), ((0, M_PAD - M_GT), (0, 0)))
    gx0, gy0, gx1, gy1 = [gpad[:, i] for i in range(4)]
    gcls = jnp.pad(gt_classes.astype(jnp.int32), (0, M_PAD - M_GT),
                   constant_values=NUM_CLASSES)

    perm = jnp.asarray(_PERM)

    iou_sc, cls_sc = _iou_cls_kernel()(
        px0, py0, px1, py1, gx0, gy0, gx1, gy1, gcls)
    iou_tc, cls_tc = _tc_iou(px0, py0, px1, py1, gx0, gy0, gx1, gy1, gcls)
    iou_pad = jnp.concatenate([iou_tc, iou_sc])
    cls_pad = jnp.concatenate([cls_tc, cls_sc])
    sampled_idxs, sampled_classes, sbox = _select_kernel()(
        cls_pad, perm, px0, py0, px1, py1)

    sampled_boxes = sbox.T
    iou_with_gt = iou_pad[:N_TOT]
    return sampled_idxs, sampled_classes, sampled_boxes, iou_with_gt


# speedup vs baseline: 1.3374x; 1.0308x over previous
import functools

import numpy as np

import jax
import jax.numpy as jnp
from jax import lax
from jax.experimental import pallas as pl
from jax.experimental.pallas import tpu as pltpu
from jax.experimental.pallas import tpu_sc as plsc

N_PROPOSALS = 20000
M_GT = 100
N_TOT = N_PROPOSALS + M_GT
NUM_CLASSES = 80
NUM_FG = 128
NUM_BG = 384
NUM_SAMPLES = NUM_FG + NUM_BG

LANES = 16
N_PAD = 20480
N_WORKERS = 32
CHUNK = N_PAD // N_WORKERS
VPC = CHUNK // LANES
M_PAD = 128

TC_ROWS = 13312
SC_ROWS = N_PAD - TC_ROWS
SC_CHUNK = SC_ROWS // N_WORKERS
SC_VPC = SC_CHUNK // LANES

_MESH = plsc.VectorSubcoreMesh(core_axis_name="c", subcore_axis_name="s")


def _sampling_perm() -> np.ndarray:
    mask = np.uint32(0xFFFFFFFF)

    def rotl(x, d):
        return ((x << np.uint32(d)) | (x >> np.uint32(32 - d))) & mask

    ks = [np.uint32(0), np.uint32(42),
          np.uint32(0) ^ np.uint32(42) ^ np.uint32(0x1BD11BDA)]
    rot = [13, 15, 26, 6, 17, 29, 16, 24]
    x0 = np.zeros(N_TOT, np.uint32) + ks[0]
    x1 = (np.arange(N_TOT, dtype=np.uint32) + ks[1]) & mask
    for i in range(5):
        for r in (rot[0:4] if i % 2 == 0 else rot[4:8]):
            x0 = (x0 + x1) & mask
            x1 = rotl(x1, r)
            x1 = x1 ^ x0
        x0 = (x0 + ks[(i + 1) % 3]) & mask
        x1 = (x1 + ks[(i + 2) % 3] + np.uint32(i + 1)) & mask
    bits = x0 ^ x1
    scores = (((bits >> np.uint32(9)) | np.uint32(0x3F800000))
              .view(np.float32) - np.float32(1.0))
    perm = np.argsort(-scores, kind="stable").astype(np.int32)
    return np.concatenate([perm, np.arange(N_TOT, N_PAD, dtype=np.int32)])


_PERM = _sampling_perm()


def _wid():
    return lax.axis_index("s") * 2 + lax.axis_index("c")


def _iou_cls_body(px0, py0, px1, py1, gx0, gy0, gx1, gy1, gcls,
                  iou_out, cls_out,
                  vx0, vy0, vx1, vy1, vg0, vg1, vg2, vg3, vgc,
                  viou, vcls):
    wid = _wid()
    base = TC_ROWS + wid * SC_CHUNK
    pltpu.sync_copy(px0.at[pl.ds(base, SC_CHUNK)], vx0)
    pltpu.sync_copy(py0.at[pl.ds(base, SC_CHUNK)], vy0)
    pltpu.sync_copy(px1.at[pl.ds(base, SC_CHUNK)], vx1)
    pltpu.sync_copy(py1.at[pl.ds(base, SC_CHUNK)], vy1)
    pltpu.sync_copy(gx0, vg0)
    pltpu.sync_copy(gy0, vg1)
    pltpu.sync_copy(gx1, vg2)
    pltpu.sync_copy(gy1, vg3)
    pltpu.sync_copy(gcls, vgc)

    def per_pair(i, _):
        sl0 = pl.ds((2 * i) * LANES, LANES)
        sl1 = pl.ds((2 * i + 1) * LANES, LANES)
        x0a = vx0[sl0]
        y0a = vy0[sl0]
        x1a = vx1[sl0]
        y1a = vy1[sl0]
        x0b = vx0[sl1]
        y0b = vy0[sl1]
        x1b = vx1[sl1]
        y1b = vy1[sl1]
        parea_a = (x1a - x0a) * (y1a - y0a)
        parea_b = (x1b - x0b) * (y1b - y0b)

        def one(a0, a1, a2, a3, garea, gc, x0, y0, x1, y1, parea,
                best, bcls):
            ltx = jnp.maximum(a0, x0)
            lty = jnp.maximum(a1, y0)
            rbx = jnp.minimum(a2, x1)
            rby = jnp.minimum(a3, y1)
            w = jnp.maximum(rbx - ltx, 0.0)
            h = jnp.maximum(rby - lty, 0.0)
            inter = w * h
            union = garea + parea - inter
            iou = jnp.where(inter > 0.0,
                            inter / jnp.maximum(union, 1e-8),
                            0.0)
            upd = iou > best
            return jnp.where(upd, iou, best), jnp.where(upd, gc, bcls)

        def per_gt(g, carry):
            best_a, bcls_a, best_b, bcls_b = carry
            gi = jnp.full((LANES,), g, dtype=jnp.int32)
            a0 = plsc.load_gather(vg0, [gi])
            a1 = plsc.load_gather(vg1, [gi])
            a2 = plsc.load_gather(vg2, [gi])
            a3 = plsc.load_gather(vg3, [gi])
            gc = plsc.load_gather(vgc, [gi])
            garea = (a2 - a0) * (a3 - a1)
            best_a, bcls_a = one(a0, a1, a2, a3, garea, gc,
                                 x0a, y0a, x1a, y1a, parea_a,
                                 best_a, bcls_a)
            best_b, bcls_b = one(a0, a1, a2, a3, garea, gc,
                                 x0b, y0b, x1b, y1b, parea_b,
                                 best_b, bcls_b)
            return best_a, bcls_a, best_b, bcls_b

        neg = jnp.full((LANES,), -1.0, dtype=jnp.float32)
        zero = jnp.zeros((LANES,), dtype=jnp.int32)
        best_a, bcls_a, best_b, bcls_b = lax.fori_loop(
            0, M_GT, per_gt, (neg, zero, neg, zero))
        bg = jnp.full((LANES,), NUM_CLASSES, dtype=jnp.int32)
        viou[sl0] = best_a
        vcls[sl0] = jnp.where(best_a >= 0.5, bcls_a, bg)
        viou[sl1] = best_b
        vcls[sl1] = jnp.where(best_b >= 0.5, bcls_b, bg)
        return 0

    lax.fori_loop(0, SC_VPC // 2, per_pair, 0)
    pltpu.sync_copy(viou, iou_out.at[pl.ds(wid * SC_CHUNK, SC_CHUNK)])
    pltpu.sync_copy(vcls, cls_out.at[pl.ds(wid * SC_CHUNK, SC_CHUNK)])


_iou_cls_kernel = functools.partial(
    pl.kernel, _iou_cls_body, mesh=_MESH,
    compiler_params=pltpu.CompilerParams(needs_layout_passes=False),
    out_type=[jax.ShapeDtypeStruct((SC_ROWS,), jnp.float32),
              jax.ShapeDtypeStruct((SC_ROWS,), jnp.int32)],
    scratch_types=[pltpu.VMEM((SC_CHUNK,), jnp.float32)] * 4
                  + [pltpu.VMEM((M_PAD,), jnp.float32)] * 4
                  + [pltpu.VMEM((M_PAD,), jnp.int32)]
                  + [pltpu.VMEM((SC_CHUNK,), jnp.float32),
                     pltpu.VMEM((SC_CHUNK,), jnp.int32)],
)


def _tc_iou_body(gx0s, gy0s, gx1s, gy1s, gcs,
                 px0b, py0b, px1b, py1b, iou_b, cls_b):
    x0 = px0b[...]
    y0 = py0b[...]
    x1 = px1b[...]
    y1 = py1b[...]
    parea = (x1 - x0) * (y1 - y0)

    def one_gt(g, best, bcls):
        a0 = gx0s[g]
        a1 = gy0s[g]
        a2 = gx1s[g]
        a3 = gy1s[g]
        gc = gcs[g]
        garea = (a2 - a0) * (a3 - a1)
        ltx = jnp.maximum(x0, a0)
        lty = jnp.maximum(y0, a1)
        rbx = jnp.minimum(x1, a2)
        rby = jnp.minimum(y1, a3)
        w = jnp.maximum(rbx - ltx, 0.0)
        h = jnp.maximum(rby - lty, 0.0)
        inter = w * h
        union = garea + parea - inter
        iou = jnp.where(inter > 0.0,
                        inter / jnp.maximum(union, 1e-8),
                        0.0)
        upd = iou > best
        return jnp.where(upd, iou, best), jnp.where(upd, gc, bcls)

    def per_gt4(q, carry):
        best, bcls = carry
        g = q * 4
        for k in range(4):
            best, bcls = one_gt(g + k, best, bcls)
        return best, bcls

    init = (jnp.full(x0.shape, -1.0, dtype=jnp.float32),
            jnp.zeros(x0.shape, dtype=jnp.int32))
    best, bcls = lax.fori_loop(0, M_GT // 4, per_gt4, init)
    iou_b[...] = best
    cls_b[...] = jnp.where(best >= 0.5, bcls, NUM_CLASSES)


def _tc_iou(px0, py0, px1, py1, gx0, gy0, gx1, gy1, gcls):
    rows = TC_ROWS // 128
    grid = rows // 8
    smem = pl.BlockSpec(memory_space=pltpu.SMEM)
    blk = pl.BlockSpec((8, 128), lambda i: (i, 0))
    iou, cls = pl.pallas_call(
        _tc_iou_body,
        grid=(grid,),
        in_specs=[smem] * 5 + [blk] * 4,
        out_specs=[blk, blk],
        out_shape=[jax.ShapeDtypeStruct((rows, 128), jnp.float32),
                   jax.ShapeDtypeStruct((rows, 128), jnp.int32)],
    )(gx0, gy0, gx1, gy1, gcls,
      px0[:TC_ROWS].reshape(rows, 128), py0[:TC_ROWS].reshape(rows, 128),
      px1[:TC_ROWS].reshape(rows, 128), py1[:TC_ROWS].reshape(rows, 128))
    return iou.reshape(TC_ROWS), cls.reshape(TC_ROWS)


def _select_body(cls_hbm, perm_hbm, px0, py0, px1, py1,
                 idx_out, scls_out, sbox_out,
                 vcls, vx0, vy0, vx1, vy1, vperm,
                 vidx, vocls, vob0, vob1, vob2, vob3, sem_c, sem_b):
    wid = _wid()

    @pl.when(wid == 0)
    def _():
        h_cls = pltpu.async_copy(cls_hbm, vcls, sem_c)
        h0 = pltpu.async_copy(px0, vx0, sem_b)
        h1 = pltpu.async_copy(py0, vy0, sem_b)
        h2 = pltpu.async_copy(px1, vx1, sem_b)
        h3 = pltpu.async_copy(py1, vy1, sem_b)
        h_cls.wait()

        zero = jnp.zeros((LANES,), jnp.int32)

        def chunk_cond(carry):
            c, nfgv, nbgv = carry
            return jnp.logical_and(
                c < N_WORKERS,
                jnp.logical_or(jnp.max(nfgv) < NUM_FG,
                               jnp.max(nbgv) < NUM_BG))

        def per_chunk(carry):
            c, nfgv, nbgv = carry
            pltpu.sync_copy(perm_hbm.at[pl.ds(c * CHUNK, CHUNK)], vperm)

            def per_vreg(j, carry2):
                nfgv, nbgv = carry2
                pv = vperm[pl.ds(j * LANES, LANES)]
                cv = plsc.load_gather(vcls, [pv])
                valid = pv < N_TOT
                fgm = jnp.logical_and(cv < NUM_CLASSES, valid)
                bgm = jnp.logical_and(cv == NUM_CLASSES, valid)
                cumf = nfgv + jnp.cumsum(fgm.astype(jnp.int32))
                cumb = nbgv + jnp.cumsum(bgm.astype(jnp.int32))
                self_f = jnp.logical_and(fgm, cumf <= NUM_FG)
                self_b = jnp.logical_and(bgm, cumb <= NUM_BG)
                plsc.store_scatter(vidx, [cumf - 1], pv, mask=self_f)
                plsc.store_scatter(vidx, [NUM_FG + cumb - 1], pv,
                                   mask=self_b)
                nfgv = nfgv + plsc.all_reduce_population_count(fgm)
                nbgv = nbgv + plsc.all_reduce_population_count(bgm)
                return nfgv, nbgv

            nfgv, nbgv = lax.fori_loop(0, VPC, per_vreg, (nfgv, nbgv))
            return c + 1, nfgv, nbgv

        _, nfgv, nbgv = lax.while_loop(
            chunk_cond, per_chunk, (jnp.int32(0), zero, zero))
        nfg = jnp.max(nfgv)
        nbg = jnp.max(nbgv)

        @pl.when(nfg < NUM_FG)
        def _():
            def fill_f(j, n):
                sl = pl.ds(j * LANES, LANES)
                cv = vcls[sl]
                iv = j * LANES + lax.iota(jnp.int32, LANES)
                m = jnp.logical_and(cv == NUM_CLASSES, iv < N_TOT)
                cum = n + jnp.cumsum(m.astype(jnp.int32))
                sel = jnp.logical_and(m, cum <= NUM_FG)
                plsc.store_scatter(vidx, [cum - 1], iv, mask=sel)
                return jnp.max(cum)

            lax.fori_loop(0, N_PAD // LANES, fill_f, nfg)

        @pl.when(nbg < NUM_BG)
        def _():
            def fill_b(j, n):
                sl = pl.ds(j * LANES, LANES)
                cv = vcls[sl]
                iv = j * LANES + lax.iota(jnp.int32, LANES)
                m = jnp.logical_and(cv < NUM_CLASSES, iv < N_TOT)
                cum = n + jnp.cumsum(m.astype(jnp.int32))
                sel = jnp.logical_and(m, cum <= NUM_BG)
                plsc.store_scatter(vidx, [NUM_FG + cum - 1], iv, mask=sel)
                return jnp.max(cum)

            lax.fori_loop(0, N_PAD // LANES, fill_b, nbg)

        h0.wait()
        h1.wait()
        h2.wait()
        h3.wait()

        def gather_out(k, _):
            sl = pl.ds(k * LANES, LANES)
            iv = vidx[sl]
            vocls[sl] = plsc.load_gather(vcls, [iv])
            vob0[sl] = plsc.load_gather(vx0, [iv])
            vob1[sl] = plsc.load_gather(vy0, [iv])
            vob2[sl] = plsc.load_gather(vx1, [iv])
            vob3[sl] = plsc.load_gather(vy1, [iv])
            return 0

        lax.fori_loop(0, NUM_SAMPLES // LANES, gather_out, 0)
        pltpu.sync_copy(vidx, idx_out)
        pltpu.sync_copy(vocls, scls_out)
        pltpu.sync_copy(vob0, sbox_out.at[0])
        pltpu.sync_copy(vob1, sbox_out.at[1])
        pltpu.sync_copy(vob2, sbox_out.at[2])
        pltpu.sync_copy(vob3, sbox_out.at[3])


_select_kernel = functools.partial(
    pl.kernel, _select_body, mesh=_MESH,
    compiler_params=pltpu.CompilerParams(needs_layout_passes=False),
    out_type=[jax.ShapeDtypeStruct((NUM_SAMPLES,), jnp.int32),
              jax.ShapeDtypeStruct((NUM_SAMPLES,), jnp.int32),
              jax.ShapeDtypeStruct((4, NUM_SAMPLES), jnp.float32)],
    scratch_types=[pltpu.VMEM((N_PAD,), jnp.int32)]
                  + [pltpu.VMEM((N_PAD,), jnp.float32)] * 4
                  + [pltpu.VMEM((CHUNK,), jnp.int32)]
                  + [pltpu.VMEM((NUM_SAMPLES,), jnp.int32)] * 2
                  + [pltpu.VMEM((NUM_SAMPLES,), jnp.float32)] * 4
                  + [pltpu.SemaphoreType.DMA] * 2,
)


def kernel(proposal_boxes, gt_boxes, gt_classes):
    props = jnp.concatenate(
        [proposal_boxes.astype(jnp.float32), gt_boxes.astype(jnp.float32)],
        axis=0)
    props_pad = jnp.pad(props, ((0, N_PAD - N_TOT), (0, 0)))
    px0, py0, px1, py1 = [props_pad[:, i] for i in range(4)]
    gpad = jnp.pad(gt_boxes.astype(jnp.float32), ((0, M_PAD - M_GT), (0, 0)))
    gx0, gy0, gx1, gy1 = [gpad[:, i] for i in range(4)]
    gcls = jnp.pad(gt_classes.astype(jnp.int32), (0, M_PAD - M_GT),
                   constant_values=NUM_CLASSES)

    perm = jnp.asarray(_PERM)

    iou_sc, cls_sc = _iou_cls_kernel()(
        px0, py0, px1, py1, gx0, gy0, gx1, gy1, gcls)
    iou_tc, cls_tc = _tc_iou(px0, py0, px1, py1, gx0, gy0, gx1, gy1, gcls)
    iou_pad = jnp.concatenate([iou_tc, iou_sc])
    cls_pad = jnp.concatenate([cls_tc, cls_sc])
    sampled_idxs, sampled_classes, sbox = _select_kernel()(
        cls_pad, perm, px0, py0, px1, py1)

    sampled_boxes = sbox.T
    iou_with_gt = iou_pad[:N_TOT]
    return sampled_idxs, sampled_classes, sampled_boxes, iou_with_gt
